# baseline (speedup 1.0000x reference)
import functools

import jax
import jax.numpy as jnp
from jax import lax
from jax.experimental import pallas as pl
from jax.experimental.pallas import tpu as pltpu
from jax.experimental.pallas import tpu_sc as plsc

NUM_EMBEDDINGS = 1000000
EMBED_DIM = 64
PADDING_IDX = 0
BATCH = 4096
SEQ = 200

NC = 2
NS = 16
NW = NC * NS
BPW = BATCH // NW

NG = 4
NT = 4
GLEAD = 3
LANES = 16
PADDED = 128

_MESH = plsc.VectorSubcoreMesh(
    core_axis_name="c", subcore_axis_name="s", num_cores=NC, num_subcores=NS
)


@functools.partial(
    pl.kernel,
    out_type=jax.ShapeDtypeStruct((SEQ, 8, NW, 8, BPW), jnp.float32),
    mesh=_MESH,
    scratch_types=[
        pltpu.VMEM((SEQ // 8, 8, BPW), jnp.int32),
        pltpu.VMEM((NG, BPW, PADDED), jnp.float32),
        pltpu.VMEM((NT, 8, 8, BPW), jnp.float32),
        pltpu.SemaphoreType.DMA((NG,)),
        pltpu.SemaphoreType.DMA((NT,)),
    ],
    compiler_params=pltpu.CompilerParams(
        use_tc_tiling_on_sc=False,
        needs_layout_passes=False,
        disable_bounds_checks=True,
    ),
)
def _gather_sc(x5_hbm, table_hbm, out_hbm, idx_v, g_v, t_v, gsem, ssem):
    wid = lax.axis_index("s") * NC + lax.axis_index("c")
    pltpu.sync_copy(x5_hbm.at[:, wid], idx_v)

    def g_desc(s, g):
        return pltpu.make_async_copy(
            table_hbm.at[idx_v.at[s // 8, s % 8]],
            g_v.at[g],
            gsem.at[g],
        )

    def s_desc(s, t):
        return pltpu.make_async_copy(
            t_v.at[t], out_hbm.at[s, :, wid], ssem.at[t]
        )

    iota = lax.iota(jnp.int32, LANES)
    rows = [iota + LANES * v for v in range(BPW // LANES)]

    def transpose(g, t):
        @plsc.parallel_loop(0, LANES, unroll=2)
        def _(o):
            rot = (iota + o) & (LANES - 1)
            for w in range(EMBED_DIM // LANES):
                c16 = rot + LANES * w
                tc16 = c16 >> 3
                ci16 = c16 & 7
                for v in range(BPW // LANES):
                    vals = plsc.load_gather(g_v.at[g], [rows[v], c16])
                    plsc.store_scatter(
                        t_v.at[t], [tc16, ci16, rows[v]], vals
                    )

    for g in range(GLEAD + 1):
        g_desc(g, g).start()

    def outer(k, carry):
        s0 = k * NG
        for i in range(NG):
            s = s0 + i
            g = i
            t = i % NT
            g_desc(s, g).wait()

            @pl.when(s >= NT)
            def _():
                s_desc(s - NT, t).wait()

            transpose(g, t)
            s_desc(s, t).start()
            j = s + GLEAD + 1

            @pl.when(j < SEQ)
            def _():
                g_desc(j, (i + GLEAD + 1) % NG).start()
        return carry

    lax.fori_loop(0, SEQ // NG, outer, 0)

    for i in range(NT):
        s = SEQ - NT + i
        s_desc(s, s % NT).wait()


TBLK = 512


def _pad_body(t_ref, o_ref):
    o_ref[:, :EMBED_DIM] = t_ref[...].T
    o_ref[:, EMBED_DIM:] = jnp.zeros(
        (TBLK, PADDED - EMBED_DIM), jnp.float32
    )


def _pad_table_tc(tT):
    grid = (NUM_EMBEDDINGS + TBLK - 1) // TBLK
    return pl.pallas_call(
        _pad_body,
        out_shape=jax.ShapeDtypeStruct((NUM_EMBEDDINGS, PADDED), jnp.float32),
        grid=(grid,),
        in_specs=[pl.BlockSpec((EMBED_DIM, TBLK), lambda i: (0, i))],
        out_specs=pl.BlockSpec((TBLK, PADDED), lambda i: (i, 0)),
    )(tT)


def _mask_body(x_ref, m_ref):
    m_ref[...] = (x_ref[...] != PADDING_IDX).astype(jnp.float32)


def _mask_tc(x5):
    return pl.pallas_call(
        _mask_body,
        out_shape=jax.ShapeDtypeStruct((SEQ // 8, NW, 8, BPW), jnp.float32),
        grid=(5,),
        in_specs=[pl.BlockSpec((5, NW, 8, BPW), lambda i: (i, 0, 0, 0))],
        out_specs=pl.BlockSpec((5, NW, 8, BPW), lambda i: (i, 0, 0, 0)),
    )(x5)


def kernel(x, table):
    x5 = x.T.reshape(SEQ // 8, 8, NW, BPW).transpose(0, 2, 1, 3)
    t2 = _pad_table_tc(table.T)
    out5 = _gather_sc(x5, t2)
    mask5 = _mask_tc(x5)
    out = out5.transpose(2, 4, 0, 1, 3).reshape(BATCH, SEQ, EMBED_DIM)
    mask = mask5.transpose(1, 3, 0, 2).reshape(BATCH, SEQ)
    return out, mask


# speedup vs baseline: 2.8811x; 2.8811x over previous
import functools

import jax
import jax.numpy as jnp
from jax import lax
from jax.experimental import pallas as pl
from jax.experimental.pallas import tpu as pltpu
from jax.experimental.pallas import tpu_sc as plsc

NUM_EMBEDDINGS = 1000000
EMBED_DIM = 64
PADDING_IDX = 0
BATCH = 4096
SEQ = 200

NC = 2
NS = 16
NW = NC * NS
BPW = BATCH // NW

NG = 4
NT = 4
GLEAD = 3
LANES = 16
PADDED = 128

_MESH = plsc.VectorSubcoreMesh(
    core_axis_name="c", subcore_axis_name="s", num_cores=NC, num_subcores=NS
)


@functools.partial(
    pl.kernel,
    out_type=jax.ShapeDtypeStruct((SEQ, 8, NW, 8, BPW), jnp.float32),
    mesh=_MESH,
    scratch_types=[
        pltpu.VMEM((SEQ // 8, 8, BPW), jnp.int32),
        pltpu.VMEM((NG, BPW, PADDED), jnp.float32),
        pltpu.VMEM((NT, 8, 8, BPW), jnp.float32),
        pltpu.SemaphoreType.DMA((NG,)),
        pltpu.SemaphoreType.DMA((NT,)),
    ],
    compiler_params=pltpu.CompilerParams(
        use_tc_tiling_on_sc=False,
        needs_layout_passes=False,
        disable_bounds_checks=True,
    ),
)
def _gather_sc(x5_hbm, table_hbm, out_hbm, idx_v, g_v, t_v, gsem, ssem):
    wid = lax.axis_index("s") * NC + lax.axis_index("c")
    pltpu.sync_copy(x5_hbm.at[:, wid], idx_v)

    def g_desc(s, g):
        return pltpu.make_async_copy(
            table_hbm.at[idx_v.at[s // 8, s % 8]],
            g_v.at[g],
            gsem.at[g],
        )

    def s_desc(s, t):
        return pltpu.make_async_copy(
            t_v.at[t], out_hbm.at[s, :, wid], ssem.at[t]
        )

    iota = lax.iota(jnp.int32, LANES)
    rows = [iota + LANES * v for v in range(BPW // LANES)]

    def transpose(g, t):
        @plsc.parallel_loop(0, LANES, unroll=2)
        def _(o):
            rot = (iota + o) & (LANES - 1)
            for w in range(EMBED_DIM // LANES):
                c16 = rot + LANES * w
                tc16 = c16 >> 3
                ci16 = c16 & 7
                for v in range(BPW // LANES):
                    vals = plsc.load_gather(g_v.at[g], [rows[v], c16])
                    plsc.store_scatter(
                        t_v.at[t], [tc16, ci16, rows[v]], vals
                    )

    for g in range(GLEAD + 1):
        g_desc(g, g).start()

    def outer(k, carry):
        s0 = k * NG
        for i in range(NG):
            s = s0 + i
            g = i
            t = i % NT
            g_desc(s, g).wait()

            @pl.when(s >= NT)
            def _():
                s_desc(s - NT, t).wait()

            transpose(g, t)
            s_desc(s, t).start()
            j = s + GLEAD + 1

            @pl.when(j < SEQ)
            def _():
                g_desc(j, (i + GLEAD + 1) % NG).start()
        return carry

    lax.fori_loop(0, SEQ // NG, outer, 0)

    for i in range(NT):
        s = SEQ - NT + i
        s_desc(s, s % NT).wait()


TBLK = 8192


def _pad_body(t_ref, o_ref):
    o_ref[:, :EMBED_DIM] = t_ref[...].T
    o_ref[:, EMBED_DIM:] = jnp.zeros(
        (TBLK, PADDED - EMBED_DIM), jnp.float32
    )


def _pad_table_tc(tT):
    grid = (NUM_EMBEDDINGS + TBLK - 1) // TBLK
    return pl.pallas_call(
        _pad_body,
        out_shape=jax.ShapeDtypeStruct((NUM_EMBEDDINGS, PADDED), jnp.float32),
        grid=(grid,),
        in_specs=[pl.BlockSpec((EMBED_DIM, TBLK), lambda i: (0, i))],
        out_specs=pl.BlockSpec((TBLK, PADDED), lambda i: (i, 0)),
    )(tT)


def _mask_body(x_ref, m_ref):
    m_ref[...] = (x_ref[...] != PADDING_IDX).astype(jnp.float32)


def _mask_tc(x5):
    return pl.pallas_call(
        _mask_body,
        out_shape=jax.ShapeDtypeStruct((SEQ // 8, NW, 8, BPW), jnp.float32),
        grid=(5,),
        in_specs=[pl.BlockSpec((5, NW, 8, BPW), lambda i: (i, 0, 0, 0))],
        out_specs=pl.BlockSpec((5, NW, 8, BPW), lambda i: (i, 0, 0, 0)),
    )(x5)


def kernel(x, table):
    x5 = x.T.reshape(SEQ // 8, 8, NW, BPW).transpose(0, 2, 1, 3)
    t2 = _pad_table_tc(table.T)
    out5 = _gather_sc(x5, t2)
    mask5 = _mask_tc(x5)
    out = out5.transpose(2, 4, 0, 1, 3).reshape(BATCH, SEQ, EMBED_DIM)
    mask = mask5.transpose(1, 3, 0, 2).reshape(BATCH, SEQ)
    return out, mask


# speedup vs baseline: 2.9966x; 1.0401x over previous
import functools

import jax
import jax.numpy as jnp
from jax import lax
from jax.experimental import pallas as pl
from jax.experimental.pallas import tpu as pltpu
from jax.experimental.pallas import tpu_sc as plsc

NUM_EMBEDDINGS = 1000000
EMBED_DIM = 64
PADDING_IDX = 0
BATCH = 4096
SEQ = 200

NC = 2
NS = 16
NW = NC * NS
BPW = BATCH // NW

NG = 4
NT = 4
GLEAD = 3
LANES = 16
PADDED = 128

_MESH = plsc.VectorSubcoreMesh(
    core_axis_name="c", subcore_axis_name="s", num_cores=NC, num_subcores=NS
)


@functools.partial(
    pl.kernel,
    out_type=jax.ShapeDtypeStruct((SEQ, 8, NW, 8, BPW), jnp.float32),
    mesh=_MESH,
    scratch_types=[
        pltpu.VMEM((SEQ // 8, 8, BPW), jnp.int32),
        pltpu.VMEM((NG, BPW, PADDED), jnp.float32),
        pltpu.VMEM((NT, 8, 8, BPW), jnp.float32),
        pltpu.SemaphoreType.DMA((NG,)),
        pltpu.SemaphoreType.DMA((NT,)),
    ],
    compiler_params=pltpu.CompilerParams(
        use_tc_tiling_on_sc=False,
        needs_layout_passes=False,
        disable_bounds_checks=True,
    ),
)
def _gather_sc(x5_hbm, table_hbm, out_hbm, idx_v, g_v, t_v, gsem, ssem):
    wid = lax.axis_index("s") * NC + lax.axis_index("c")
    pltpu.sync_copy(x5_hbm.at[:, wid], idx_v)

    def g_desc(s, g):
        return pltpu.make_async_copy(
            table_hbm.at[idx_v.at[s // 8, s % 8]],
            g_v.at[g],
            gsem.at[g],
        )

    def s_desc(s, t):
        return pltpu.make_async_copy(
            t_v.at[t], out_hbm.at[s, :, wid], ssem.at[t]
        )

    iota = lax.iota(jnp.int32, LANES)
    rows = [iota + LANES * v for v in range(BPW // LANES)]

    def transpose(g, t):
        @plsc.parallel_loop(0, LANES, unroll=2)
        def _(o):
            rot = (iota + o) & (LANES - 1)
            for w in range(EMBED_DIM // LANES):
                c16 = rot + LANES * w
                tc16 = c16 >> 3
                ci16 = c16 & 7
                for v in range(BPW // LANES):
                    vals = plsc.load_gather(g_v.at[g], [rows[v], c16])
                    plsc.store_scatter(
                        t_v.at[t], [tc16, ci16, rows[v]], vals
                    )

    for g in range(GLEAD + 1):
        g_desc(g, g).start()

    def outer(k, carry):
        s0 = k * NG
        for i in range(NG):
            s = s0 + i
            g = i
            t = i % NT
            g_desc(s, g).wait()

            @pl.when(s >= NT)
            def _():
                s_desc(s - NT, t).wait()

            transpose(g, t)
            s_desc(s, t).start()
            j = s + GLEAD + 1

            @pl.when(j < SEQ)
            def _():
                g_desc(j, (i + GLEAD + 1) % NG).start()
        return carry

    lax.fori_loop(0, SEQ // NG, outer, 0)

    for i in range(NT):
        s = SEQ - NT + i
        s_desc(s, s % NT).wait()


TBLK = 16384


def _pad_body(t_ref, o_ref):
    o_ref[:, :EMBED_DIM] = t_ref[...].T
    o_ref[:, EMBED_DIM:] = jnp.zeros(
        (TBLK, PADDED - EMBED_DIM), jnp.float32
    )


def _pad_table_tc(tT):
    grid = (NUM_EMBEDDINGS + TBLK - 1) // TBLK
    return pl.pallas_call(
        _pad_body,
        out_shape=jax.ShapeDtypeStruct((NUM_EMBEDDINGS, PADDED), jnp.float32),
        grid=(grid,),
        in_specs=[pl.BlockSpec((EMBED_DIM, TBLK), lambda i: (0, i))],
        out_specs=pl.BlockSpec((TBLK, PADDED), lambda i: (i, 0)),
    )(tT)


def _mask_body(x_ref, m_ref):
    m_ref[...] = (x_ref[...] != PADDING_IDX).astype(jnp.float32)


def _mask_tc(x5):
    return pl.pallas_call(
        _mask_body,
        out_shape=jax.ShapeDtypeStruct((SEQ // 8, NW, 8, BPW), jnp.float32),
        grid=(5,),
        in_specs=[pl.BlockSpec((5, NW, 8, BPW), lambda i: (i, 0, 0, 0))],
        out_specs=pl.BlockSpec((5, NW, 8, BPW), lambda i: (i, 0, 0, 0)),
    )(x5)


def kernel(x, table):
    x5 = x.T.reshape(SEQ // 8, 8, NW, BPW).transpose(0, 2, 1, 3)
    t2 = _pad_table_tc(table.T)
    out5 = _gather_sc(x5, t2)
    mask5 = _mask_tc(x5)
    out = out5.transpose(2, 4, 0, 1, 3).reshape(BATCH, SEQ, EMBED_DIM)
    mask = mask5.transpose(1, 3, 0, 2).reshape(BATCH, SEQ)
    return out, mask
